# baseline (speedup 1.0000x reference)
import functools

import jax
import jax.numpy as jnp
from jax import lax
from jax.experimental import pallas as pl
from jax.experimental.pallas import tpu as pltpu
from jax.experimental.pallas import tpu_sc as plsc

_NSEG = 128
_D = 128
_EN = 8192
_EL = 4096
_R = 512
_NT = _EN // _R
_CH = 128
_BIG = 1e38
_SENT = -1e38


def _sc_gather(h, node_edge, label_edge):
    info = plsc.get_sparse_core_info()
    nc, ns = info.num_cores, info.num_subcores
    nw = nc * ns
    n_chunks_n = _EN // (_CH * nw)
    n_chunks_l = _EL // (_CH * nw)

    mesh = plsc.VectorSubcoreMesh(core_axis_name="c", subcore_axis_name="s")

    @functools.partial(
        pl.kernel,
        out_type=(
            jax.ShapeDtypeStruct((2, _EN, _D), jnp.float32),
            jax.ShapeDtypeStruct((2, _EL, _D), jnp.float32),
        ),
        mesh=mesh,
        scratch_types=[
            pltpu.VMEM((_CH,), jnp.int32),
            pltpu.VMEM((_CH, _D), jnp.float32),
            pltpu.SemaphoreType.DMA,
        ],
    )
    def k(h_hbm, ne_hbm, le_hbm, gn_hbm, gl_hbm, idx_v, rows_v, sem):
        wid = lax.axis_index("s") * nc + lax.axis_index("c")
        for j in range(2):
            for c in range(n_chunks_n):
                base = pl.multiple_of((wid * n_chunks_n + c) * _CH, _CH)
                pltpu.sync_copy(ne_hbm.at[j, pl.ds(base, _CH)], idx_v)
                pltpu.async_copy(h_hbm.at[idx_v], rows_v, sem).wait()
                pltpu.sync_copy(rows_v, gn_hbm.at[j, pl.ds(base, _CH)])
            for c in range(n_chunks_l):
                base = pl.multiple_of((wid * n_chunks_l + c) * _CH, _CH)
                pltpu.sync_copy(le_hbm.at[j, pl.ds(base, _CH)], idx_v)
                pltpu.async_copy(h_hbm.at[idx_v], rows_v, sem).wait()
                pltpu.sync_copy(rows_v, gl_hbm.at[j, pl.ds(base, _CH)])

    return k(h, node_edge, label_edge)


def _tc_body(gn_ref, gl_ref, lab_ref, nst_ref, nsf_ref, nsc_ref, lsc_ref,
             out_ref, acc1, m2):
    i = pl.program_id(0)
    nt = pl.num_programs(0)
    f32 = jnp.float32

    bf16 = jnp.bfloat16
    dd = (((1,), (1,)), ((), ()))
    a = (gn_ref[0] + gn_ref[1]) * 0.5
    b = (gl_ref[0] + gl_ref[1]) * 0.5
    a2 = jnp.sum(a * a, axis=1, keepdims=True)
    ah = a.astype(bf16)
    al = (a - ah.astype(f32)).astype(bf16)
    bh = b.astype(bf16)
    bl = (b - bh.astype(f32)).astype(bf16)
    lhs3 = jnp.concatenate([ah, ah, al], axis=1)
    rhs3 = jnp.concatenate([bh, bl, bh], axis=1)
    ab = lax.dot_general(lhs3, rhs3, dd, preferred_element_type=f32)
    bsq = b * b
    bsqh = bsq.astype(bf16)
    bsql = (bsq - bsqh.astype(f32)).astype(bf16)
    onesb = jnp.ones((1, _D), bf16)
    b2 = (lax.dot_general(onesb, bsqh, dd, preferred_element_type=f32)
          + lax.dot_general(onesb, bsql, dd, preferred_element_type=f32))
    b2c = jnp.sum(bsq, axis=1, keepdims=True)
    d2f = a2 + b2 - 2.0 * ab

    abt = lax.dot_general(rhs3, lhs3, dd, preferred_element_type=f32)
    asq = a * a
    asqh = asq.astype(bf16)
    asql = (asq - asqh.astype(f32)).astype(bf16)
    a2r = (lax.dot_general(onesb, asqh, dd, preferred_element_type=f32)
           + lax.dot_general(onesb, asql, dd, preferred_element_type=f32))
    d2t = a2r + b2c - 2.0 * abt

    lab = lab_ref[...]
    iota_l = lax.broadcasted_iota(jnp.int32, (1, _EL), 1)
    lsc = lsc_ref[...]

    scan1 = d2t
    d = 1
    while d < _EL:
        lscr = pltpu.roll(lsc, d, axis=0)
        valid1 = lsc == lscr
        cand1 = jnp.where(valid1, pltpu.roll(scan1, d, axis=0), _BIG)
        scan1 = jnp.minimum(scan1, cand1)
        d *= 2

    labn = pltpu.roll(lab, _EL - 1, axis=1)
    is_last = (lab != labn) | (iota_l >= _EL - 1)
    gseg_l = lax.broadcasted_iota(jnp.int32, (_NSEG, _EL), 0)
    g1t = jnp.where((gseg_l == lab) & is_last, 1.0, 0.0)
    min1 = lax.dot_general(g1t, scan1, (((1,), (0,)), ((), ())),
                           preferred_element_type=f32)
    lab_present = jnp.sum(g1t, axis=1, keepdims=True)
    d2_1 = jnp.maximum(min1, 0.0)
    m1 = jnp.where(lab_present > 0.0, -jnp.sqrt(d2_1), 0.0)

    nst = nst_ref[0]
    eqn = lax.broadcasted_iota(jnp.int32, (_NSEG, _R), 0) == nst
    eqnf = eqn.astype(f32)
    contrib = lax.dot_general(eqnf, m1, (((1,), (1,)), ((), ())),
                              preferred_element_type=f32)

    @pl.when(i == 0)
    def _():
        acc1[...] = contrib

    @pl.when(i > 0)
    def _():
        acc1[...] = acc1[...] + contrib

    nsc = nsc_ref[...]
    iota_s = lax.broadcasted_iota(jnp.int32, (_R, 1), 0)
    scan2 = d2f
    d = 1
    while d < _R:
        nscr = pltpu.roll(nsc, d, axis=0)
        valid2 = (nsc == nscr) & (iota_s >= d)
        cand2 = jnp.where(valid2, pltpu.roll(scan2, d, axis=0), _BIG)
        scan2 = jnp.minimum(scan2, cand2)
        d *= 2

    nstn = pltpu.roll(nst, _R - 1, axis=1)
    iota_r = lax.broadcasted_iota(jnp.int32, (1, _R), 1)
    is_last2 = (nst != nstn) | (iota_r >= _R - 1)
    g2 = jnp.where(eqn & is_last2, 1.0, 0.0)
    ext = lax.dot_general(g2, scan2, (((1,), (0,)), ((), ())),
                          preferred_element_type=f32)
    present = jnp.sum(eqnf, axis=1, keepdims=True) > 0.0
    extm = jnp.where(present, ext, _BIG)

    @pl.when(i == 0)
    def _():
        m2[...] = extm

    @pl.when(i > 0)
    def _():
        m2[...] = jnp.minimum(m2[...], extm)

    @pl.when(i == nt - 1)
    def _():
        nsf = nsf_ref[...]
        eqf = (lax.broadcasted_iota(jnp.int32, (_NSEG, _EN), 0) == nsf
               ).astype(f32)
        cn = jnp.sum(eqf, axis=1, keepdims=True)
        out1 = acc1[...] / jnp.maximum(cn, 1.0)
        m2v = m2[...]
        d2_2 = jnp.maximum(m2v, 0.0)
        m2m = jnp.where(m2v >= _BIG * 0.5, 0.0, -jnp.sqrt(d2_2))
        eql = (gseg_l == lab).astype(f32)
        cl = jnp.sum(eql, axis=1, keepdims=True)
        wlt = eql / jnp.maximum(cl, 1.0)
        out2 = lax.dot_general(m2m, wlt, (((1,), (1,)), ((), ())),
                               preferred_element_type=f32)
        out_ref[...] = (out1 + out2) * 0.5


def _tc_call(gn, gl, lab, nst3, nsf, nsc, lsc, interpret=False):
    return pl.pallas_call(
        _tc_body,
        grid=(_NT,),
        in_specs=[
            pl.BlockSpec((2, _R, _D), lambda i: (0, i, 0)),
            pl.BlockSpec((2, _EL, _D), lambda i: (0, 0, 0)),
            pl.BlockSpec((1, _EL), lambda i: (0, 0)),
            pl.BlockSpec((1, 1, _R), lambda i: (i, 0, 0)),
            pl.BlockSpec((1, _EN), lambda i: (0, 0)),
            pl.BlockSpec((_R, 1), lambda i: (i, 0)),
            pl.BlockSpec((_EL, 1), lambda i: (0, 0)),
        ],
        out_specs=pl.BlockSpec((_NSEG, _NSEG), lambda i: (0, 0)),
        out_shape=jax.ShapeDtypeStruct((_NSEG, _NSEG), jnp.float32),
        scratch_shapes=[
            pltpu.VMEM((_NSEG, _NSEG), jnp.float32),
            pltpu.VMEM((_NSEG, _EL), jnp.float32),
        ],
        interpret=interpret,
    )(gn, gl, lab, nst3, nsf, nsc, lsc)


def kernel(h, node_edge, node_seg, label_edge, label_seg):
    gn, gl = _sc_gather(h, node_edge, label_edge)
    lab = label_seg.reshape(1, _EL)
    nst3 = node_seg.reshape(_NT, 1, _R)
    nsf = node_seg.reshape(1, _EN)
    nsc = node_seg.reshape(_EN, 1)
    lsc = label_seg.reshape(_EL, 1)
    return _tc_call(gn, gl, lab, nst3, nsf, nsc, lsc)


# speedup vs baseline: 1.0509x; 1.0509x over previous
import functools

import jax
import jax.numpy as jnp
from jax import lax
from jax.experimental import pallas as pl
from jax.experimental.pallas import tpu as pltpu
from jax.experimental.pallas import tpu_sc as plsc

_NSEG = 128
_D = 128
_EN = 8192
_EL = 4096
_R = 512
_NT = _EN // _R
_CH = 128
_LC = 128
_NLC = _EL // _LC
_BIG = 1e38
_SENT = -1e38


def _sc_gather(h, node_edge, label_edge):
    info = plsc.get_sparse_core_info()
    nc, ns = info.num_cores, info.num_subcores
    nw = nc * ns
    n_chunks_n = _EN // (_CH * nw)
    n_chunks_l = _EL // (_CH * nw)

    mesh = plsc.VectorSubcoreMesh(core_axis_name="c", subcore_axis_name="s")

    @functools.partial(
        pl.kernel,
        out_type=(
            jax.ShapeDtypeStruct((2, _EN, _D), jnp.float32),
            jax.ShapeDtypeStruct((2, _EL, _D), jnp.float32),
        ),
        mesh=mesh,
        scratch_types=[
            pltpu.VMEM((_CH,), jnp.int32),
            pltpu.VMEM((_CH, _D), jnp.float32),
            pltpu.SemaphoreType.DMA,
        ],
    )
    def k(h_hbm, ne_hbm, le_hbm, gn_hbm, gl_hbm, idx_v, rows_v, sem):
        wid = lax.axis_index("s") * nc + lax.axis_index("c")
        for j in range(2):
            for c in range(n_chunks_n):
                base = pl.multiple_of((wid * n_chunks_n + c) * _CH, _CH)
                pltpu.sync_copy(ne_hbm.at[j, pl.ds(base, _CH)], idx_v)
                pltpu.async_copy(h_hbm.at[idx_v], rows_v, sem).wait()
                pltpu.sync_copy(rows_v, gn_hbm.at[j, pl.ds(base, _CH)])
            for c in range(n_chunks_l):
                base = pl.multiple_of((wid * n_chunks_l + c) * _CH, _CH)
                pltpu.sync_copy(le_hbm.at[j, pl.ds(base, _CH)], idx_v)
                pltpu.async_copy(h_hbm.at[idx_v], rows_v, sem).wait()
                pltpu.sync_copy(rows_v, gl_hbm.at[j, pl.ds(base, _CH)])

    return k(h, node_edge, label_edge)


def _tc_body(gn_ref, gl_ref, lab_ref, nst_ref, nsf_ref, nsc_ref,
             out_ref, acc1, m2, rhs3_s, b2_s, g1t_s, hsel_s, bcast_s,
             gatec_s, cok_s, lp_s):
    i = pl.program_id(0)
    nt = pl.num_programs(0)
    f32 = jnp.float32
    bf16 = jnp.bfloat16
    dd = (((1,), (1,)), ((), ()))

    lab = lab_ref[...]
    iota_l = lax.broadcasted_iota(jnp.int32, (1, _EL), 1)

    @pl.when(i == 0)
    def _():
        b = (gl_ref[0] + gl_ref[1]) * 0.5
        bh = b.astype(bf16)
        bl = (b - bh.astype(f32)).astype(bf16)
        rhs3_s[...] = jnp.concatenate([bh, bl, bh], axis=1)
        bsq = b * b
        bsqh = bsq.astype(bf16)
        bsql = (bsq - bsqh.astype(f32)).astype(bf16)
        onesb = jnp.ones((1, _D), bf16)
        b2_s[...] = (lax.dot_general(onesb, bsqh, dd, preferred_element_type=f32)
                     + lax.dot_general(onesb, bsql, dd,
                                       preferred_element_type=f32))
        labn = pltpu.roll(lab, _EL - 1, axis=1)
        is_last = (lab != labn) | (iota_l >= _EL - 1)
        gseg_l = lax.broadcasted_iota(jnp.int32, (_NSEG, _EL), 0)
        g1t = jnp.where((gseg_l == lab) & is_last, 1.0, 0.0)
        g1t_s[...] = g1t
        lp_s[...] = lax.dot_general(jnp.ones((1, _EL), f32), g1t,
                                    (((1,), (1,)), ((), ())),
                                    preferred_element_type=f32)
        jd = lax.broadcasted_iota(jnp.int32, (_EL, _NLC), 0)
        kd = lax.broadcasted_iota(jnp.int32, (_EL, _NLC), 1)
        hsel_s[...] = jnp.where((jd // _LC == kd) & (jd % _LC == _LC - 1),
                                1.0, 0.0)
        hsel_st = jnp.where((jd // _LC == kd) & (jd % _LC == 0), 1.0, 0.0)
        kb = lax.broadcasted_iota(jnp.int32, (_NLC, _EL), 0)
        jb = lax.broadcasted_iota(jnp.int32, (_NLC, _EL), 1)
        bcast = jnp.where(kb == jb // _LC, 1.0, 0.0)
        bcast_s[...] = bcast
        lab_f = lab.astype(f32)
        ids_e = lax.dot_general(lab_f, hsel_s[...], (((1,), (0,)), ((), ())),
                                preferred_element_type=f32)
        ids_s = lax.dot_general(lab_f, hsel_st, (((1,), (0,)), ((), ())),
                                preferred_element_type=f32)
        iota_c = lax.broadcasted_iota(jnp.int32, (1, _NLC), 1)
        cok_s[...] = jnp.where(
            (pltpu.roll(ids_e, 1, axis=1) == ids_s) & (iota_c >= 1), 1.0, 0.0)
        ss_b = lax.dot_general(ids_s, bcast, (((1,), (0,)), ((), ())),
                               preferred_element_type=f32)
        gatec_s[...] = jnp.where(lab_f == ss_b, 1.0, 0.0)

    a = (gn_ref[0] + gn_ref[1]) * 0.5
    a2 = jnp.sum(a * a, axis=1, keepdims=True)
    ah = a.astype(bf16)
    al = (a - ah.astype(f32)).astype(bf16)
    lhs3 = jnp.concatenate([ah, ah, al], axis=1)
    ab = lax.dot_general(lhs3, rhs3_s[...], dd, preferred_element_type=f32)
    d2f = a2 + b2_s[...] - 2.0 * ab

    scan = d2f
    d = 1
    while d < _LC:
        labr = pltpu.roll(lab, d, axis=1)
        valid = (lab == labr) & (iota_l >= d)
        cand = jnp.where(valid, pltpu.roll(scan, d, axis=1), _BIG)
        scan = jnp.minimum(scan, cand)
        d *= 2
    lsum = lax.dot_general(scan, hsel_s[...], (((1,), (0,)), ((), ())),
                           preferred_element_type=f32)
    lab_f = lab.astype(f32)
    ids_e = lax.dot_general(lab_f, hsel_s[...], (((1,), (0,)), ((), ())),
                            preferred_element_type=f32)
    iota_c = lax.broadcasted_iota(jnp.int32, (1, _NLC), 1)
    d = 1
    while d < _NLC:
        idr = pltpu.roll(ids_e, d, axis=1)
        validc = (ids_e == idr) & (iota_c >= d)
        candc = jnp.where(validc, pltpu.roll(lsum, d, axis=1), _BIG)
        lsum = jnp.minimum(lsum, candc)
        d *= 2
    pcar = jnp.where(cok_s[...] > 0.0, pltpu.roll(lsum, 1, axis=1), _BIG)
    pcol = lax.dot_general(pcar, bcast_s[...], (((1,), (0,)), ((), ())),
                           preferred_element_type=f32)
    final1 = jnp.minimum(scan, jnp.where(gatec_s[...] > 0.0, pcol, _BIG))

    min1 = lax.dot_general(final1, g1t_s[...], (((1,), (1,)), ((), ())),
                           preferred_element_type=f32)
    d2_1 = jnp.maximum(min1, 0.0)
    m1 = jnp.where(lp_s[...] > 0.0, -jnp.sqrt(d2_1), 0.0)

    nst = nst_ref[0]
    eqn = lax.broadcasted_iota(jnp.int32, (_NSEG, _R), 0) == nst
    eqnf = eqn.astype(f32)
    contrib = lax.dot_general(eqnf, m1, (((1,), (0,)), ((), ())),
                              preferred_element_type=f32)

    @pl.when(i == 0)
    def _():
        acc1[...] = contrib

    @pl.when(i > 0)
    def _():
        acc1[...] = acc1[...] + contrib

    nsc = nsc_ref[...]
    iota_s = lax.broadcasted_iota(jnp.int32, (_R, 1), 0)
    scan2 = d2f
    d = 1
    while d < _R:
        nscr = pltpu.roll(nsc, d, axis=0)
        valid2 = (nsc == nscr) & (iota_s >= d)
        cand2 = jnp.where(valid2, pltpu.roll(scan2, d, axis=0), _BIG)
        scan2 = jnp.minimum(scan2, cand2)
        d *= 2

    nstn = pltpu.roll(nst, _R - 1, axis=1)
    iota_r = lax.broadcasted_iota(jnp.int32, (1, _R), 1)
    is_last2 = (nst != nstn) | (iota_r >= _R - 1)
    g2 = jnp.where(eqn & is_last2, 1.0, 0.0)
    ext = lax.dot_general(g2, scan2, (((1,), (0,)), ((), ())),
                          preferred_element_type=f32)
    present = jnp.sum(eqnf, axis=1, keepdims=True) > 0.0
    extm = jnp.where(present, ext, _BIG)

    @pl.when(i == 0)
    def _():
        m2[...] = extm

    @pl.when(i > 0)
    def _():
        m2[...] = jnp.minimum(m2[...], extm)

    @pl.when(i == nt - 1)
    def _():
        nsf = nsf_ref[...]
        eqf = (lax.broadcasted_iota(jnp.int32, (_NSEG, _EN), 0) == nsf
               ).astype(f32)
        cn = jnp.sum(eqf, axis=1, keepdims=True)
        out1 = acc1[...] / jnp.maximum(cn, 1.0)
        m2v = m2[...]
        d2_2 = jnp.maximum(m2v, 0.0)
        m2m = jnp.where(m2v >= _BIG * 0.5, 0.0, -jnp.sqrt(d2_2))
        gseg_l = lax.broadcasted_iota(jnp.int32, (_NSEG, _EL), 0)
        eql = (gseg_l == lab).astype(f32)
        cl = jnp.sum(eql, axis=1, keepdims=True)
        wlt = eql / jnp.maximum(cl, 1.0)
        out2 = lax.dot_general(m2m, wlt, (((1,), (1,)), ((), ())),
                               preferred_element_type=f32)
        out_ref[...] = (out1 + out2) * 0.5


def _tc_call(gn, gl, lab, nst3, nsf, nsc, interpret=False):
    return pl.pallas_call(
        _tc_body,
        grid=(_NT,),
        in_specs=[
            pl.BlockSpec((2, _R, _D), lambda i: (0, i, 0)),
            pl.BlockSpec((2, _EL, _D), lambda i: (0, 0, 0)),
            pl.BlockSpec((1, _EL), lambda i: (0, 0)),
            pl.BlockSpec((1, 1, _R), lambda i: (i, 0, 0)),
            pl.BlockSpec((1, _EN), lambda i: (0, 0)),
            pl.BlockSpec((_R, 1), lambda i: (i, 0)),
        ],
        out_specs=pl.BlockSpec((_NSEG, _NSEG), lambda i: (0, 0)),
        out_shape=jax.ShapeDtypeStruct((_NSEG, _NSEG), jnp.float32),
        scratch_shapes=[
            pltpu.VMEM((_NSEG, _NSEG), jnp.float32),
            pltpu.VMEM((_NSEG, _EL), jnp.float32),
            pltpu.VMEM((_EL, 3 * _D), jnp.bfloat16),
            pltpu.VMEM((1, _EL), jnp.float32),
            pltpu.VMEM((_NSEG, _EL), jnp.float32),
            pltpu.VMEM((_EL, _NLC), jnp.float32),
            pltpu.VMEM((_NLC, _EL), jnp.float32),
            pltpu.VMEM((1, _EL), jnp.float32),
            pltpu.VMEM((1, _NLC), jnp.float32),
            pltpu.VMEM((1, _NSEG), jnp.float32),
        ],
        interpret=interpret,
    )(gn, gl, lab, nst3, nsf, nsc)


def kernel(h, node_edge, node_seg, label_edge, label_seg):
    gn, gl = _sc_gather(h, node_edge, label_edge)
    lab = label_seg.reshape(1, _EL)
    nst3 = node_seg.reshape(_NT, 1, _R)
    nsf = node_seg.reshape(1, _EN)
    nsc = node_seg.reshape(_EN, 1)
    return _tc_call(gn, gl, lab, nst3, nsf, nsc)


# speedup vs baseline: 1.5025x; 1.4297x over previous
import functools

import jax
import jax.numpy as jnp
from jax import lax
from jax.experimental import pallas as pl
from jax.experimental.pallas import tpu as pltpu
from jax.experimental.pallas import tpu_sc as plsc

_NSEG = 128
_D = 128
_EN = 8192
_EL = 4096
_R = 512
_NT = _EN // _R
_CH = 128
_NEG = float("-inf")


def _sc_gather(h, node_edge, label_edge):
    info = plsc.get_sparse_core_info()
    nc, ns = info.num_cores, info.num_subcores
    nw = nc * ns
    n_chunks_n = _EN // (_CH * nw)
    n_chunks_l = _EL // (_CH * nw)

    mesh = plsc.VectorSubcoreMesh(core_axis_name="c", subcore_axis_name="s")

    @functools.partial(
        pl.kernel,
        out_type=(
            jax.ShapeDtypeStruct((2, _EN, _D), jnp.float32),
            jax.ShapeDtypeStruct((2, _EL, _D), jnp.float32),
        ),
        mesh=mesh,
        scratch_types=[
            pltpu.VMEM((_CH,), jnp.int32),
            pltpu.VMEM((_CH, _D), jnp.float32),
            pltpu.SemaphoreType.DMA,
        ],
    )
    def k(h_hbm, ne_hbm, le_hbm, gn_hbm, gl_hbm, idx_v, rows_v, sem):
        wid = lax.axis_index("s") * nc + lax.axis_index("c")
        for j in range(2):
            for c in range(n_chunks_n):
                base = pl.multiple_of((wid * n_chunks_n + c) * _CH, _CH)
                pltpu.sync_copy(ne_hbm.at[j, pl.ds(base, _CH)], idx_v)
                pltpu.async_copy(h_hbm.at[idx_v], rows_v, sem).wait()
                pltpu.sync_copy(rows_v, gn_hbm.at[j, pl.ds(base, _CH)])
            for c in range(n_chunks_l):
                base = pl.multiple_of((wid * n_chunks_l + c) * _CH, _CH)
                pltpu.sync_copy(le_hbm.at[j, pl.ds(base, _CH)], idx_v)
                pltpu.async_copy(h_hbm.at[idx_v], rows_v, sem).wait()
                pltpu.sync_copy(rows_v, gl_hbm.at[j, pl.ds(base, _CH)])

    return k(h, node_edge, label_edge)


def _tc_body(gn_ref, gl_ref, lab_ref, nst_ref, nsf_ref, nsc_ref,
             out_ref, acc1, m2):
    i = pl.program_id(0)
    nt = pl.num_programs(0)
    f32 = jnp.float32

    a = (gn_ref[0] + gn_ref[1]) * 0.5
    b = (gl_ref[0] + gl_ref[1]) * 0.5
    a2 = jnp.sum(a * a, axis=1, keepdims=True)
    ones = jnp.ones((1, _D), f32)
    b2 = lax.dot_general(ones, b * b, (((1,), (1,)), ((), ())),
                         preferred_element_type=f32)
    ab = lax.dot_general(a, b, (((1,), (1,)), ((), ())),
                         preferred_element_type=f32)
    d2 = jnp.maximum(a2 + b2 - 2.0 * ab, 0.0)
    edge = -jnp.sqrt(d2)

    lab = lab_ref[...]
    iota_l = lax.broadcasted_iota(jnp.int32, (1, _EL), 1)

    scan = edge
    d = 1
    while d < _EL:
        labr = pltpu.roll(lab, d, axis=1)
        valid = (lab == labr) & (iota_l >= d)
        cand = jnp.where(valid, pltpu.roll(scan, d, axis=1), _NEG)
        scan = jnp.maximum(scan, cand)
        d *= 2

    labn = pltpu.roll(lab, _EL - 1, axis=1)
    is_last = (lab != labn) | (iota_l >= _EL - 1)
    gseg_l = lax.broadcasted_iota(jnp.int32, (_NSEG, _EL), 0)
    g1t = jnp.where((gseg_l == lab) & is_last, 1.0, 0.0)
    m1 = lax.dot_general(scan, g1t, (((1,), (1,)), ((), ())),
                         preferred_element_type=f32)

    nst = nst_ref[0]
    eqn = lax.broadcasted_iota(jnp.int32, (_NSEG, _R), 0) == nst
    eqnf = eqn.astype(f32)
    contrib = jnp.dot(eqnf, m1, preferred_element_type=f32)

    @pl.when(i == 0)
    def _():
        acc1[...] = contrib

    @pl.when(i > 0)
    def _():
        acc1[...] = acc1[...] + contrib

    nsc = nsc_ref[...]
    iota_s = lax.broadcasted_iota(jnp.int32, (_R, 1), 0)
    scan2 = edge
    d = 1
    while d < _R:
        nscr = pltpu.roll(nsc, d, axis=0)
        valid2 = (nsc == nscr) & (iota_s >= d)
        cand2 = jnp.where(valid2, pltpu.roll(scan2, d, axis=0), _NEG)
        scan2 = jnp.maximum(scan2, cand2)
        d *= 2

    nstn = pltpu.roll(nst, _R - 1, axis=1)
    iota_r = lax.broadcasted_iota(jnp.int32, (1, _R), 1)
    is_last2 = (nst != nstn) | (iota_r >= _R - 1)
    g2 = jnp.where(eqn & is_last2, 1.0, 0.0)
    ext = jnp.dot(g2, scan2, preferred_element_type=f32)
    present = jnp.sum(eqnf, axis=1, keepdims=True) > 0.0
    extm = jnp.where(present, ext, _NEG)

    @pl.when(i == 0)
    def _():
        m2[...] = extm

    @pl.when(i > 0)
    def _():
        m2[...] = jnp.maximum(m2[...], extm)

    @pl.when(i == nt - 1)
    def _():
        nsf = nsf_ref[...]
        eqf = (lax.broadcasted_iota(jnp.int32, (_NSEG, _EN), 0) == nsf
               ).astype(f32)
        cn = jnp.sum(eqf, axis=1, keepdims=True)
        out1 = acc1[...] / jnp.maximum(cn, 1.0)
        m2v = m2[...]
        m2m = jnp.where(m2v == _NEG, 0.0, m2v)
        eql = (lax.broadcasted_iota(jnp.int32, (_NSEG, _EL), 0) == lab
               ).astype(f32)
        cl = jnp.sum(eql, axis=1, keepdims=True)
        wlt = eql / jnp.maximum(cl, 1.0)
        out2 = lax.dot_general(m2m, wlt, (((1,), (1,)), ((), ())),
                               preferred_element_type=f32)
        out_ref[...] = (out1 + out2) * 0.5


def _tc_call(gn, gl, lab, nst3, nsf, nsc, interpret=False):
    return pl.pallas_call(
        _tc_body,
        grid=(_NT,),
        in_specs=[
            pl.BlockSpec((2, _R, _D), lambda i: (0, i, 0)),
            pl.BlockSpec((2, _EL, _D), lambda i: (0, 0, 0)),
            pl.BlockSpec((1, _EL), lambda i: (0, 0)),
            pl.BlockSpec((1, 1, _R), lambda i: (i, 0, 0)),
            pl.BlockSpec((1, _EN), lambda i: (0, 0)),
            pl.BlockSpec((_R, 1), lambda i: (i, 0)),
        ],
        out_specs=pl.BlockSpec((_NSEG, _NSEG), lambda i: (0, 0)),
        out_shape=jax.ShapeDtypeStruct((_NSEG, _NSEG), jnp.float32),
        scratch_shapes=[
            pltpu.VMEM((_NSEG, _NSEG), jnp.float32),
            pltpu.VMEM((_NSEG, _EL), jnp.float32),
        ],
        interpret=interpret,
    )(gn, gl, lab, nst3, nsf, nsc)


def kernel(h, node_edge, node_seg, label_edge, label_seg):
    gn, gl = _sc_gather(h, node_edge, label_edge)
    lab = label_seg.reshape(1, _EL)
    nst3 = node_seg.reshape(_NT, 1, _R)
    nsf = node_seg.reshape(1, _EN)
    nsc = node_seg.reshape(_EN, 1)
    return _tc_call(gn, gl, lab, nst3, nsf, nsc)
